# baseline (speedup 1.0000x reference)
import functools

import jax
import jax.numpy as jnp
from jax import lax
from jax.experimental import pallas as pl
from jax.experimental.pallas import tpu as pltpu
from jax.experimental.pallas import tpu_sc as plsc

_N = 10000
_E = 320000
_IN = 128
_D = 64
_OUT = 40
_L = 3
_AW = 64
_DW = 16
_NC = 2
_NS = 16
_NT = _NC * _NS
_EPT = _E // _NT
_CB = 80
_NCHUNK = _EPT // _CB
_NBUF = 5
_NP = 10240
_RPT = _NP // _NS
_ZR = 128
_NB = 1000
_NG = _N // _NB


_TC_PARAMS = pltpu.CompilerParams(vmem_limit_bytes=64 * 1024 * 1024)


def _prologue(x, gw, asv, adv, hw, hb, lw, lb,
              haug_ref, aa_ref, hpre_ref, lin_ref, m_ref):
    i = pl.program_id(0)
    h = jnp.dot(x, gw, preferred_element_type=jnp.float32)
    asrc = jnp.sum(h * asv[None, :], axis=1)
    adst = jnp.sum(h * adv[None, :], axis=1)
    aa_ref[...] = jnp.concatenate([asrc, adst])[None, None, :]
    haug_ref[...] = h
    hpre_ref[...] = jnp.maximum(
        jnp.dot(x, hw, preferred_element_type=jnp.float32) + hb[None, :], 0.0)
    lin_ref[...] = jnp.dot(x, lw, preferred_element_type=jnp.float32) + lb[None, :]
    cur = jnp.concatenate([jnp.full((1, 16), jnp.max(asrc), jnp.float32),
                           jnp.full((1, 16), jnp.max(adst), jnp.float32)], 0)
    m_ref[...] = jnp.where(i == 0, cur, jnp.maximum(m_ref[...], cur))


def _epilogue(p_ref, dp_ref, hpre, lin, lnw, lnb, beta_l, xl):
    num = p_ref[0] + p_ref[1]
    den = dp_ref[0, :, 0:1] + dp_ref[1, :, 0:1]
    gat = num / (den + 1e-16)
    xn = jnp.maximum(gat + lin, 0.0)
    t = hpre * xn
    mu = jnp.mean(t, axis=1, keepdims=True)
    var = jnp.mean((t - mu) * (t - mu), axis=1, keepdims=True)
    ln = (t - mu) / jnp.sqrt(var + 1e-5) * lnw[None, :] + lnb[None, :]
    beta = jax.nn.sigmoid(beta_l)[None, :]
    x = (1.0 - beta) * ln + beta * xn
    return x, xl + x


def _first_body(x_ref, inw_ref, inb_ref, gw_ref, asv_ref, adv_ref, hw_ref,
                hb_ref, lw_ref, lb_ref,
                haug_ref, aa_ref, hpre_ref, lin_ref, m_ref, x_out_ref):
    x = jnp.dot(x_ref[...], inw_ref[...],
                preferred_element_type=jnp.float32) + inb_ref[...][None, :]
    x_out_ref[...] = x
    _prologue(x, gw_ref[...], asv_ref[...], adv_ref[...], hw_ref[...],
              hb_ref[...], lw_ref[...], lb_ref[...],
              haug_ref, aa_ref, hpre_ref, lin_ref, m_ref)


def _make_mid_body(first):
    def _mid_body(p_ref, dp_ref, hpre_ref, lin_ref, lnw_ref, lnb_ref, beta_ref,
                  xl_ref, gw_ref, asv_ref, adv_ref, hw_ref, hb_ref, lw_ref,
                  lb_ref, haug_ref, aa_ref, hpre_out_ref, lin_out_ref, m_ref,
                  xl_out_ref):
        xl_in = jnp.zeros((_NB, _D), jnp.float32) if first else xl_ref[...]
        x, xl = _epilogue(p_ref, dp_ref, hpre_ref[...], lin_ref[...],
                          lnw_ref[...], lnb_ref[...], beta_ref[...], xl_in)
        xl_out_ref[...] = xl
        _prologue(x, gw_ref[...], asv_ref[...], adv_ref[...], hw_ref[...],
                  hb_ref[...], lw_ref[...], lb_ref[...],
                  haug_ref, aa_ref, hpre_out_ref, lin_out_ref, m_ref)
    return _mid_body


_mid_bodies = [_make_mid_body(True), _make_mid_body(False)]


def _last_body(p_ref, dp_ref, hpre_ref, lin_ref, lnw_ref, lnb_ref, beta_ref,
               xl_ref, pw_ref, pb_ref, o_ref):
    _, xl = _epilogue(p_ref, dp_ref, hpre_ref[...], lin_ref[...], lnw_ref[...],
                      lnb_ref[...], beta_ref[...], xl_ref[...])
    o_ref[...] = jnp.dot(xl, pw_ref[...],
                         preferred_element_type=jnp.float32) + pb_ref[...][None, :]


def _sc_edge_body(aa_h, m_h, src_h, dst_h, haug_h, out_h, den_h,
                  asrc_v, adst_v, m_v, srcv, dstv, denr, rows, acc, dacc,
                  gsem, ssem, dsem):
    c = lax.axis_index("c")
    s = lax.axis_index("s")
    wid = c * _NS + s

    @pl.loop(0, _CB)
    def _zero(r):
        for cc in range(_AW // 16):
            rows[0][r, pl.ds(cc * 16, 16)] = jnp.zeros((16,), jnp.float32)

    for b in range(_NBUF):
        @pl.loop(0, _CB)
        def _zden(r):
            denr[b][r, pl.ds(0, 16)] = jnp.zeros((16,), jnp.float32)

    for z in range(_RPT // _CB):
        pltpu.sync_copy(rows[0], acc.at[pl.ds(s * _RPT + z * _CB, _CB)])
        pltpu.sync_copy(denr[0], dacc.at[pl.ds(s * _RPT + z * _CB, _CB)])

    for r in range(_NG):
        pltpu.sync_copy(aa_h.at[r, 0, pl.ds(0, _NB)],
                        asrc_v.at[pl.ds(r * _NB, _NB)])
        pltpu.sync_copy(aa_h.at[r, 0, pl.ds(_NB, _NB)],
                        adst_v.at[pl.ds(r * _NB, _NB)])
    pltpu.sync_copy(m_h, m_v)
    pltpu.sync_copy(src_h.at[wid], srcv)
    pltpu.sync_copy(dst_h.at[wid], dstv)
    plsc.subcore_barrier()

    mm = m_v[0, pl.ds(0, 16)] + m_v[1, pl.ds(0, 16)]
    m = jnp.where(mm > 0.0, mm, 0.2 * mm)
    iota16 = lax.iota(jnp.int32, 16)
    zero16 = jnp.zeros((16,), jnp.int32)

    for b in range(_NBUF):
        pltpu.async_copy(haug_h.at[srcv.at[b]], rows[b], gsem.at[b])

    @pl.loop(0, _NCHUNK, step=_NBUF)
    def _group(g):
        for b in range(_NBUF):
            k = g + b
            for j in range(_CB // 16):
                si = srcv[k, pl.ds(j * 16, 16)]
                di = dstv[k, pl.ds(j * 16, 16)]
                a = (plsc.load_gather(asrc_v, [si])
                     + plsc.load_gather(adst_v, [di]))
                la = jnp.where(a > 0.0, a, 0.2 * a)
                plsc.store_scatter(denr[b], [iota16 + j * 16, zero16],
                                   jnp.exp(la - m))

            pltpu.async_copy(denr[b], dacc.at[dstv.at[k]], dsem.at[b],
                             add=True)

            pltpu.make_async_copy(haug_h.at[srcv.at[k]], rows[b],
                                  gsem.at[b]).wait()

            @plsc.parallel_loop(0, _CB, unroll=4)
            def _row(r):
                sp = plsc.load_gather(denr[b], [jnp.full((16,), r, jnp.int32),
                                                zero16])
                for cc in range(_AW // 16):
                    rows[b][r, pl.ds(cc * 16, 16)] = \
                        rows[b][r, pl.ds(cc * 16, 16)] * sp

            pltpu.async_copy(rows[b], acc.at[dstv.at[k]], ssem.at[b],
                             add=True)

            @pl.when(g < _NCHUNK - _NBUF)
            def _prefetch():
                pltpu.make_async_copy(rows[b], acc.at[dstv.at[k]],
                                      ssem.at[b]).wait()
                pltpu.async_copy(haug_h.at[srcv.at[k + _NBUF]], rows[b],
                                 gsem.at[b])
                pltpu.make_async_copy(denr[b], dacc.at[dstv.at[k]],
                                      dsem.at[b]).wait()

    for b in range(_NBUF):
        k = _NCHUNK - _NBUF + b
        pltpu.make_async_copy(rows[b], acc.at[dstv.at[k]], ssem.at[b]).wait()
        pltpu.make_async_copy(denr[b], dacc.at[dstv.at[k]], dsem.at[b]).wait()

    plsc.subcore_barrier()
    for z in range(_RPT // _ZR):
        r0 = s * _RPT + z * _ZR
        pltpu.sync_copy(acc.at[pl.ds(r0, _ZR)], out_h.at[c, pl.ds(r0, _ZR)])
        pltpu.sync_copy(dacc.at[pl.ds(r0, _ZR)], den_h.at[c, pl.ds(r0, _ZR)])


_sc_edge = functools.partial(
    pl.kernel,
    out_type=(jax.ShapeDtypeStruct((_NC, _NP, _AW), jnp.float32),
              jax.ShapeDtypeStruct((_NC, _NP, _DW), jnp.float32)),
    mesh=plsc.VectorSubcoreMesh(core_axis_name="c", subcore_axis_name="s"),
    compiler_params=pltpu.CompilerParams(needs_layout_passes=False,
                                         use_tc_tiling_on_sc=False),
    scratch_types=[
        pltpu.VMEM((_N,), jnp.float32),
        pltpu.VMEM((_N,), jnp.float32),
        pltpu.VMEM((2, 16), jnp.float32),
        pltpu.VMEM((_NCHUNK, _CB), jnp.int32),
        pltpu.VMEM((_NCHUNK, _CB), jnp.int32),
        [pltpu.VMEM((_CB, _DW), jnp.float32)] * _NBUF,
        [pltpu.VMEM((_CB, _AW), jnp.float32)] * _NBUF,
        pltpu.VMEM_SHARED((_NP, _AW), jnp.float32),
        pltpu.VMEM_SHARED((_NP, _DW), jnp.float32),
        pltpu.SemaphoreType.DMA((_NBUF,)),
        pltpu.SemaphoreType.DMA((_NBUF,)),
        pltpu.SemaphoreType.DMA((_NBUF,)),
    ],
)(_sc_edge_body)


def kernel(x, edge_index, lin_in_W, lin_in_b, h_lin_W, h_lin_b, lin_W, lin_b,
           gat_W, att_src, att_dst, ln_w, ln_b, betas, pred_W, pred_b):
    src3 = edge_index[0].reshape(_NT, _NCHUNK, _CB)
    dst3 = edge_index[1].reshape(_NT, _NCHUNK, _CB)
    f32 = jnp.float32

    bs_nd = pl.BlockSpec((_NB, _D), lambda i: (i, 0))
    bs_aa = pl.BlockSpec((1, 1, 2 * _NB), lambda i: (i, 0, 0))
    bs_p = pl.BlockSpec((_NC, _NB, _AW), lambda i: (0, i, 0))
    bs_dp = pl.BlockSpec((_NC, _NB, _DW), lambda i: (0, i, 0))
    bs_w = pl.BlockSpec((_D, _D), lambda i: (0, 0))
    bs_v = pl.BlockSpec((_D,), lambda i: (0,))
    bs_m = pl.BlockSpec((2, 16), lambda i: (0, 0))

    node_shapes = [
        jax.ShapeDtypeStruct((_N, _D), f32),
        jax.ShapeDtypeStruct((_NG, 1, 2 * _NB), f32),
        jax.ShapeDtypeStruct((_N, _D), f32),
        jax.ShapeDtypeStruct((_N, _D), f32),
        jax.ShapeDtypeStruct((2, 16), f32),
    ]
    node_specs = [bs_nd, bs_aa, bs_nd, bs_nd, bs_m]
    pro_w_specs = [bs_w, bs_v, bs_v, bs_w, bs_v, bs_w, bs_v]

    haug, aa, hpre, lin, m, _ = pl.pallas_call(
        _first_body,
        grid=(_NG,),
        in_specs=[pl.BlockSpec((_NB, _IN), lambda i: (i, 0)),
                  pl.BlockSpec((_IN, _D), lambda i: (0, 0)), bs_v]
                 + pro_w_specs,
        out_specs=node_specs + [bs_nd],
        out_shape=node_shapes + [jax.ShapeDtypeStruct((_N, _D), f32)],
        compiler_params=_TC_PARAMS,
    )(x, lin_in_W, lin_in_b, gat_W[0], att_src[0].reshape(-1),
      att_dst[0].reshape(-1), h_lin_W[0], h_lin_b[0], lin_W[0], lin_b[0])

    xl = hpre
    for i in range(_L - 1):
        p, dp = _sc_edge(aa, m, src3, dst3, haug)
        haug, aa, hpre, lin, m, xl = pl.pallas_call(
            _mid_bodies[0 if i == 0 else 1],
            grid=(_NG,),
            in_specs=[bs_p, bs_dp, bs_nd, bs_nd, bs_v, bs_v, bs_v, bs_nd]
                     + pro_w_specs,
            out_specs=node_specs + [bs_nd],
            out_shape=node_shapes + [jax.ShapeDtypeStruct((_N, _D), f32)],
            compiler_params=_TC_PARAMS,
        )(p, dp, hpre, lin, ln_w[i], ln_b[i], betas[i], xl,
          gat_W[i + 1], att_src[i + 1].reshape(-1), att_dst[i + 1].reshape(-1),
          h_lin_W[i + 1], h_lin_b[i + 1], lin_W[i + 1], lin_b[i + 1])

    p, dp = _sc_edge(aa, m, src3, dst3, haug)
    out = pl.pallas_call(
        _last_body,
        grid=(_NG,),
        in_specs=[bs_p, bs_dp, bs_nd, bs_nd, bs_v, bs_v, bs_v, bs_nd,
                  pl.BlockSpec((_D, _OUT), lambda i: (0, 0)),
                  pl.BlockSpec((_OUT,), lambda i: (0,))],
        out_specs=pl.BlockSpec((_NB, _OUT), lambda i: (i, 0)),
        out_shape=jax.ShapeDtypeStruct((_N, _OUT), f32),
        compiler_params=_TC_PARAMS,
    )(p, dp, hpre, lin, ln_w[_L - 1], ln_b[_L - 1], betas[_L - 1], xl,
      pred_W, pred_b)
    return out


# speedup vs baseline: 1.1574x; 1.1574x over previous
import functools

import jax
import jax.numpy as jnp
from jax import lax
from jax.experimental import pallas as pl
from jax.experimental.pallas import tpu as pltpu
from jax.experimental.pallas import tpu_sc as plsc

_N = 10000
_E = 320000
_IN = 128
_D = 64
_OUT = 40
_L = 3
_AW = 64
_DW = 16
_NC = 2
_NS = 16
_NT = _NC * _NS
_EPT = _E // _NT
_CB = 80
_NCHUNK = _EPT // _CB
_NBUF = 5
_NP = 10240
_RPT = _NP // _NS
_ZR = 128
_NB = 2000
_NG = _N // _NB


_TC_PARAMS = pltpu.CompilerParams(vmem_limit_bytes=64 * 1024 * 1024)


def _prologue(x, gw, asv, adv, hw, hb, lw, lb,
              haug_ref, aa_ref, hpre_ref, lin_ref, m_ref):
    i = pl.program_id(0)
    h = jnp.dot(x, gw, preferred_element_type=jnp.float32)
    asrc = jnp.sum(h * asv[None, :], axis=1)
    adst = jnp.sum(h * adv[None, :], axis=1)
    aa_ref[...] = jnp.concatenate([asrc, adst])[None, None, :]
    haug_ref[...] = h
    hpre_ref[...] = jnp.maximum(
        jnp.dot(x, hw, preferred_element_type=jnp.float32) + hb[None, :], 0.0)
    lin_ref[...] = jnp.dot(x, lw, preferred_element_type=jnp.float32) + lb[None, :]
    cur = jnp.concatenate([jnp.full((1, 16), jnp.max(asrc), jnp.float32),
                           jnp.full((1, 16), jnp.max(adst), jnp.float32)], 0)
    m_ref[...] = jnp.where(i == 0, cur, jnp.maximum(m_ref[...], cur))


def _epilogue(p_ref, dp_ref, hpre, lin, lnw, lnb, beta_l, xl):
    num = p_ref[0] + p_ref[1]
    den = dp_ref[0, :, 0:1] + dp_ref[1, :, 0:1]
    gat = num / (den + 1e-16)
    xn = jnp.maximum(gat + lin, 0.0)
    t = hpre * xn
    mu = jnp.mean(t, axis=1, keepdims=True)
    var = jnp.mean((t - mu) * (t - mu), axis=1, keepdims=True)
    ln = (t - mu) / jnp.sqrt(var + 1e-5) * lnw[None, :] + lnb[None, :]
    beta = jax.nn.sigmoid(beta_l)[None, :]
    x = (1.0 - beta) * ln + beta * xn
    return x, xl + x


def _first_body(x_ref, inw_ref, inb_ref, gw_ref, asv_ref, adv_ref, hw_ref,
                hb_ref, lw_ref, lb_ref,
                haug_ref, aa_ref, hpre_ref, lin_ref, m_ref, x_out_ref):
    x = jnp.dot(x_ref[...], inw_ref[...],
                preferred_element_type=jnp.float32) + inb_ref[...][None, :]
    x_out_ref[...] = x
    _prologue(x, gw_ref[...], asv_ref[...], adv_ref[...], hw_ref[...],
              hb_ref[...], lw_ref[...], lb_ref[...],
              haug_ref, aa_ref, hpre_ref, lin_ref, m_ref)


def _make_mid_body(first):
    def _mid_body(p_ref, dp_ref, hpre_ref, lin_ref, lnw_ref, lnb_ref, beta_ref,
                  xl_ref, gw_ref, asv_ref, adv_ref, hw_ref, hb_ref, lw_ref,
                  lb_ref, haug_ref, aa_ref, hpre_out_ref, lin_out_ref, m_ref,
                  xl_out_ref):
        xl_in = jnp.zeros((_NB, _D), jnp.float32) if first else xl_ref[...]
        x, xl = _epilogue(p_ref, dp_ref, hpre_ref[...], lin_ref[...],
                          lnw_ref[...], lnb_ref[...], beta_ref[...], xl_in)
        xl_out_ref[...] = xl
        _prologue(x, gw_ref[...], asv_ref[...], adv_ref[...], hw_ref[...],
                  hb_ref[...], lw_ref[...], lb_ref[...],
                  haug_ref, aa_ref, hpre_out_ref, lin_out_ref, m_ref)
    return _mid_body


_mid_bodies = [_make_mid_body(True), _make_mid_body(False)]


def _last_body(p_ref, dp_ref, hpre_ref, lin_ref, lnw_ref, lnb_ref, beta_ref,
               xl_ref, pw_ref, pb_ref, o_ref):
    _, xl = _epilogue(p_ref, dp_ref, hpre_ref[...], lin_ref[...], lnw_ref[...],
                      lnb_ref[...], beta_ref[...], xl_ref[...])
    o_ref[...] = jnp.dot(xl, pw_ref[...],
                         preferred_element_type=jnp.float32) + pb_ref[...][None, :]


def _sc_edge_body(aa_h, m_h, src_h, dst_h, haug_h, out_h, den_h,
                  asrc_v, adst_v, m_v, srcv, dstv, denr, rows, acc, dacc,
                  gsem, ssem, dsem):
    c = lax.axis_index("c")
    s = lax.axis_index("s")
    wid = c * _NS + s

    in_copies = []
    for r in range(_NG):
        in_copies.append((aa_h.at[r, 0, pl.ds(0, _NB)],
                          asrc_v.at[pl.ds(r * _NB, _NB)]))
        in_copies.append((aa_h.at[r, 0, pl.ds(_NB, _NB)],
                          adst_v.at[pl.ds(r * _NB, _NB)]))
    in_copies += [(m_h, m_v), (src_h.at[wid], srcv), (dst_h.at[wid], dstv)]
    for src, dst in in_copies:
        pltpu.async_copy(src, dst, gsem.at[0])

    @pl.loop(0, _CB)
    def _zero(r):
        for cc in range(_AW // 16):
            rows[0][r, pl.ds(cc * 16, 16)] = jnp.zeros((16,), jnp.float32)

    for b in range(_NBUF):
        @pl.loop(0, _CB)
        def _zden(r):
            denr[b][r, pl.ds(0, 16)] = jnp.zeros((16,), jnp.float32)

    for z in range(_RPT // _CB):
        pltpu.sync_copy(rows[0], acc.at[pl.ds(s * _RPT + z * _CB, _CB)])
        pltpu.sync_copy(denr[0], dacc.at[pl.ds(s * _RPT + z * _CB, _CB)])

    for src, dst in in_copies:
        pltpu.make_async_copy(src, dst, gsem.at[0]).wait()
    plsc.subcore_barrier()

    mm = m_v[0, pl.ds(0, 16)] + m_v[1, pl.ds(0, 16)]
    m = jnp.where(mm > 0.0, mm, 0.2 * mm)
    iota16 = lax.iota(jnp.int32, 16)
    zero16 = jnp.zeros((16,), jnp.int32)

    for b in range(_NBUF):
        pltpu.async_copy(haug_h.at[srcv.at[b]], rows[b], gsem.at[b])

    @pl.loop(0, _NCHUNK, step=_NBUF)
    def _group(g):
        for b in range(_NBUF):
            k = g + b
            for j in range(_CB // 16):
                si = srcv[k, pl.ds(j * 16, 16)]
                di = dstv[k, pl.ds(j * 16, 16)]
                a = (plsc.load_gather(asrc_v, [si])
                     + plsc.load_gather(adst_v, [di]))
                la = jnp.where(a > 0.0, a, 0.2 * a)
                plsc.store_scatter(denr[b], [iota16 + j * 16, zero16],
                                   jnp.exp(la - m))

            pltpu.async_copy(denr[b], dacc.at[dstv.at[k]], dsem.at[b],
                             add=True)

            pltpu.make_async_copy(haug_h.at[srcv.at[k]], rows[b],
                                  gsem.at[b]).wait()

            @plsc.parallel_loop(0, _CB, unroll=4)
            def _row(r):
                sp = plsc.load_gather(denr[b], [jnp.full((16,), r, jnp.int32),
                                                zero16])
                for cc in range(_AW // 16):
                    rows[b][r, pl.ds(cc * 16, 16)] = \
                        rows[b][r, pl.ds(cc * 16, 16)] * sp

            pltpu.async_copy(rows[b], acc.at[dstv.at[k]], ssem.at[b],
                             add=True)

            @pl.when(g < _NCHUNK - _NBUF)
            def _prefetch():
                pltpu.make_async_copy(rows[b], acc.at[dstv.at[k]],
                                      ssem.at[b]).wait()
                pltpu.async_copy(haug_h.at[srcv.at[k + _NBUF]], rows[b],
                                 gsem.at[b])
                pltpu.make_async_copy(denr[b], dacc.at[dstv.at[k]],
                                      dsem.at[b]).wait()

    for b in range(_NBUF):
        k = _NCHUNK - _NBUF + b
        pltpu.make_async_copy(rows[b], acc.at[dstv.at[k]], ssem.at[b]).wait()
        pltpu.make_async_copy(denr[b], dacc.at[dstv.at[k]], dsem.at[b]).wait()

    plsc.subcore_barrier()
    for z in range(_RPT // _ZR):
        r0 = s * _RPT + z * _ZR
        pltpu.sync_copy(acc.at[pl.ds(r0, _ZR)], out_h.at[c, pl.ds(r0, _ZR)])
        pltpu.sync_copy(dacc.at[pl.ds(r0, _ZR)], den_h.at[c, pl.ds(r0, _ZR)])


_sc_edge = functools.partial(
    pl.kernel,
    out_type=(jax.ShapeDtypeStruct((_NC, _NP, _AW), jnp.float32),
              jax.ShapeDtypeStruct((_NC, _NP, _DW), jnp.float32)),
    mesh=plsc.VectorSubcoreMesh(core_axis_name="c", subcore_axis_name="s"),
    compiler_params=pltpu.CompilerParams(needs_layout_passes=False,
                                         use_tc_tiling_on_sc=False),
    scratch_types=[
        pltpu.VMEM((_N,), jnp.float32),
        pltpu.VMEM((_N,), jnp.float32),
        pltpu.VMEM((2, 16), jnp.float32),
        pltpu.VMEM((_NCHUNK, _CB), jnp.int32),
        pltpu.VMEM((_NCHUNK, _CB), jnp.int32),
        [pltpu.VMEM((_CB, _DW), jnp.float32)] * _NBUF,
        [pltpu.VMEM((_CB, _AW), jnp.float32)] * _NBUF,
        pltpu.VMEM_SHARED((_NP, _AW), jnp.float32),
        pltpu.VMEM_SHARED((_NP, _DW), jnp.float32),
        pltpu.SemaphoreType.DMA((_NBUF,)),
        pltpu.SemaphoreType.DMA((_NBUF,)),
        pltpu.SemaphoreType.DMA((_NBUF,)),
    ],
)(_sc_edge_body)


def kernel(x, edge_index, lin_in_W, lin_in_b, h_lin_W, h_lin_b, lin_W, lin_b,
           gat_W, att_src, att_dst, ln_w, ln_b, betas, pred_W, pred_b):
    src3 = edge_index[0].reshape(_NT, _NCHUNK, _CB)
    dst3 = edge_index[1].reshape(_NT, _NCHUNK, _CB)
    f32 = jnp.float32

    bs_nd = pl.BlockSpec((_NB, _D), lambda i: (i, 0))
    bs_aa = pl.BlockSpec((1, 1, 2 * _NB), lambda i: (i, 0, 0))
    bs_p = pl.BlockSpec((_NC, _NB, _AW), lambda i: (0, i, 0))
    bs_dp = pl.BlockSpec((_NC, _NB, _DW), lambda i: (0, i, 0))
    bs_w = pl.BlockSpec((_D, _D), lambda i: (0, 0))
    bs_v = pl.BlockSpec((_D,), lambda i: (0,))
    bs_m = pl.BlockSpec((2, 16), lambda i: (0, 0))

    node_shapes = [
        jax.ShapeDtypeStruct((_N, _D), f32),
        jax.ShapeDtypeStruct((_NG, 1, 2 * _NB), f32),
        jax.ShapeDtypeStruct((_N, _D), f32),
        jax.ShapeDtypeStruct((_N, _D), f32),
        jax.ShapeDtypeStruct((2, 16), f32),
    ]
    node_specs = [bs_nd, bs_aa, bs_nd, bs_nd, bs_m]
    pro_w_specs = [bs_w, bs_v, bs_v, bs_w, bs_v, bs_w, bs_v]

    haug, aa, hpre, lin, m, _ = pl.pallas_call(
        _first_body,
        grid=(_NG,),
        in_specs=[pl.BlockSpec((_NB, _IN), lambda i: (i, 0)),
                  pl.BlockSpec((_IN, _D), lambda i: (0, 0)), bs_v]
                 + pro_w_specs,
        out_specs=node_specs + [bs_nd],
        out_shape=node_shapes + [jax.ShapeDtypeStruct((_N, _D), f32)],
        compiler_params=_TC_PARAMS,
    )(x, lin_in_W, lin_in_b, gat_W[0], att_src[0].reshape(-1),
      att_dst[0].reshape(-1), h_lin_W[0], h_lin_b[0], lin_W[0], lin_b[0])

    xl = hpre
    for i in range(_L - 1):
        p, dp = _sc_edge(aa, m, src3, dst3, haug)
        haug, aa, hpre, lin, m, xl = pl.pallas_call(
            _mid_bodies[0 if i == 0 else 1],
            grid=(_NG,),
            in_specs=[bs_p, bs_dp, bs_nd, bs_nd, bs_v, bs_v, bs_v, bs_nd]
                     + pro_w_specs,
            out_specs=node_specs + [bs_nd],
            out_shape=node_shapes + [jax.ShapeDtypeStruct((_N, _D), f32)],
            compiler_params=_TC_PARAMS,
        )(p, dp, hpre, lin, ln_w[i], ln_b[i], betas[i], xl,
          gat_W[i + 1], att_src[i + 1].reshape(-1), att_dst[i + 1].reshape(-1),
          h_lin_W[i + 1], h_lin_b[i + 1], lin_W[i + 1], lin_b[i + 1])

    p, dp = _sc_edge(aa, m, src3, dst3, haug)
    out = pl.pallas_call(
        _last_body,
        grid=(_NG,),
        in_specs=[bs_p, bs_dp, bs_nd, bs_nd, bs_v, bs_v, bs_v, bs_nd,
                  pl.BlockSpec((_D, _OUT), lambda i: (0, 0)),
                  pl.BlockSpec((_OUT,), lambda i: (0,))],
        out_specs=pl.BlockSpec((_NB, _OUT), lambda i: (i, 0)),
        out_shape=jax.ShapeDtypeStruct((_N, _OUT), f32),
        compiler_params=_TC_PARAMS,
    )(p, dp, hpre, lin, ln_w[_L - 1], ln_b[_L - 1], betas[_L - 1], xl,
      pred_W, pred_b)
    return out


# speedup vs baseline: 1.1796x; 1.0192x over previous
import functools

import jax
import jax.numpy as jnp
from jax import lax
from jax.experimental import pallas as pl
from jax.experimental.pallas import tpu as pltpu
from jax.experimental.pallas import tpu_sc as plsc

_N = 10000
_E = 320000
_IN = 128
_D = 64
_OUT = 40
_L = 3
_AW = 64
_DW = 16
_NC = 2
_NS = 16
_NT = _NC * _NS
_EPT = _E // _NT
_CB = 80
_NCHUNK = _EPT // _CB
_NBUF = 5
_NP = 10240
_RPT = _NP // _NS
_ZR = 128
_NB = 2000
_NG = _N // _NB


_TC_PARAMS = pltpu.CompilerParams(vmem_limit_bytes=64 * 1024 * 1024)


def _prologue(x, gw, asv, adv, hw, hb, lw, lb,
              haug_ref, aa_ref, hpre_ref, lin_ref, m_ref):
    i = pl.program_id(0)
    h = jnp.dot(x, gw, preferred_element_type=jnp.float32)
    asrc = jnp.sum(h * asv[None, :], axis=1)
    adst = jnp.sum(h * adv[None, :], axis=1)
    aa_ref[...] = jnp.concatenate([asrc, adst])[None, None, :]
    haug_ref[...] = h
    hpre_ref[...] = jnp.maximum(
        jnp.dot(x, hw, preferred_element_type=jnp.float32) + hb[None, :], 0.0)
    lin_ref[...] = jnp.dot(x, lw, preferred_element_type=jnp.float32) + lb[None, :]
    cur = jnp.concatenate([jnp.full((1, 16), jnp.max(asrc), jnp.float32),
                           jnp.full((1, 16), jnp.max(adst), jnp.float32)], 0)
    m_ref[...] = jnp.where(i == 0, cur, jnp.maximum(m_ref[...], cur))


def _epilogue(p_ref, dp_ref, hpre, lin, lnw, lnb, beta_l, xl):
    num = p_ref[0] + p_ref[1]
    den = dp_ref[0, :, 0:1] + dp_ref[1, :, 0:1]
    gat = num / (den + 1e-16)
    xn = jnp.maximum(gat + lin, 0.0)
    t = hpre * xn
    mu = jnp.mean(t, axis=1, keepdims=True)
    var = jnp.mean((t - mu) * (t - mu), axis=1, keepdims=True)
    ln = (t - mu) / jnp.sqrt(var + 1e-5) * lnw[None, :] + lnb[None, :]
    beta = jax.nn.sigmoid(beta_l)[None, :]
    x = (1.0 - beta) * ln + beta * xn
    return x, xl + x


def _first_body(x_ref, inw_ref, inb_ref, gw_ref, asv_ref, adv_ref, hw_ref,
                hb_ref, lw_ref, lb_ref,
                haug_ref, aa_ref, hpre_ref, lin_ref, m_ref, x_out_ref):
    x = jnp.dot(x_ref[...], inw_ref[...],
                preferred_element_type=jnp.float32) + inb_ref[...][None, :]
    x_out_ref[...] = x
    _prologue(x, gw_ref[...], asv_ref[...], adv_ref[...], hw_ref[...],
              hb_ref[...], lw_ref[...], lb_ref[...],
              haug_ref, aa_ref, hpre_ref, lin_ref, m_ref)


def _make_mid_body(first):
    def _mid_body(p_ref, dp_ref, hpre_ref, lin_ref, lnw_ref, lnb_ref, beta_ref,
                  xl_ref, gw_ref, asv_ref, adv_ref, hw_ref, hb_ref, lw_ref,
                  lb_ref, haug_ref, aa_ref, hpre_out_ref, lin_out_ref, m_ref,
                  xl_out_ref):
        xl_in = jnp.zeros((_NB, _D), jnp.float32) if first else xl_ref[...]
        x, xl = _epilogue(p_ref, dp_ref, hpre_ref[...], lin_ref[...],
                          lnw_ref[...], lnb_ref[...], beta_ref[...], xl_in)
        xl_out_ref[...] = xl
        _prologue(x, gw_ref[...], asv_ref[...], adv_ref[...], hw_ref[...],
                  hb_ref[...], lw_ref[...], lb_ref[...],
                  haug_ref, aa_ref, hpre_out_ref, lin_out_ref, m_ref)
    return _mid_body


_mid_bodies = [_make_mid_body(True), _make_mid_body(False)]


def _last_body(p_ref, dp_ref, hpre_ref, lin_ref, lnw_ref, lnb_ref, beta_ref,
               xl_ref, pw_ref, pb_ref, o_ref):
    _, xl = _epilogue(p_ref, dp_ref, hpre_ref[...], lin_ref[...], lnw_ref[...],
                      lnb_ref[...], beta_ref[...], xl_ref[...])
    o_ref[...] = jnp.dot(xl, pw_ref[...],
                         preferred_element_type=jnp.float32) + pb_ref[...][None, :]


def _sc_edge_body(aa_h, m_h, src_h, dst_h, haug_h, out_h, den_h,
                  asrc_v, adst_v, m_v, srcv, dstv, denr, rows, acc, dacc,
                  gsem, ssem, dsem):
    c = lax.axis_index("c")
    s = lax.axis_index("s")
    wid = c * _NS + s

    in_copies = []
    for r in range(_NG):
        in_copies.append((aa_h.at[r, 0, pl.ds(0, _NB)],
                          asrc_v.at[pl.ds(r * _NB, _NB)]))
        in_copies.append((aa_h.at[r, 0, pl.ds(_NB, _NB)],
                          adst_v.at[pl.ds(r * _NB, _NB)]))
    in_copies += [(m_h, m_v), (src_h.at[wid], srcv), (dst_h.at[wid], dstv)]
    for src, dst in in_copies:
        pltpu.async_copy(src, dst, gsem.at[0])

    @pl.loop(0, _CB)
    def _zero(r):
        for cc in range(_AW // 16):
            rows[0][r, pl.ds(cc * 16, 16)] = jnp.zeros((16,), jnp.float32)

    for b in range(_NBUF):
        @pl.loop(0, _CB)
        def _zden(r):
            denr[b][r, pl.ds(0, 16)] = jnp.zeros((16,), jnp.float32)

    for z in range(_RPT // _CB):
        pltpu.sync_copy(rows[0], acc.at[pl.ds(s * _RPT + z * _CB, _CB)])
        pltpu.sync_copy(denr[0], dacc.at[pl.ds(s * _RPT + z * _CB, _CB)])

    for src, dst in in_copies:
        pltpu.make_async_copy(src, dst, gsem.at[0]).wait()
    plsc.subcore_barrier()

    mm = m_v[0, pl.ds(0, 16)] + m_v[1, pl.ds(0, 16)]
    m = jnp.where(mm > 0.0, mm, 0.2 * mm)
    iota16 = lax.iota(jnp.int32, 16)
    zero16 = jnp.zeros((16,), jnp.int32)

    for b in range(_NBUF):
        pltpu.async_copy(haug_h.at[srcv.at[b]], rows[b], gsem.at[b])

    @pl.loop(0, _NCHUNK, step=_NBUF)
    def _group(g):
        for b in range(_NBUF):
            k = g + b
            for j in range(_CB // 16):
                si = srcv[k, pl.ds(j * 16, 16)]
                di = dstv[k, pl.ds(j * 16, 16)]
                a = (plsc.load_gather(asrc_v, [si])
                     + plsc.load_gather(adst_v, [di]))
                la = jnp.where(a > 0.0, a, 0.2 * a)
                plsc.store_scatter(denr[b], [iota16 + j * 16, zero16],
                                   jnp.exp(la - m))

            pltpu.async_copy(denr[b], dacc.at[dstv.at[k]], dsem.at[b],
                             add=True)

            pltpu.make_async_copy(haug_h.at[srcv.at[k]], rows[b],
                                  gsem.at[b]).wait()

            @plsc.parallel_loop(0, _CB, unroll=4)
            def _row(r):
                sp = plsc.load_gather(denr[b], [jnp.full((16,), r, jnp.int32),
                                                zero16])
                for cc in range(_AW // 16):
                    rows[b][r, pl.ds(cc * 16, 16)] = \
                        rows[b][r, pl.ds(cc * 16, 16)] * sp

            pltpu.async_copy(rows[b], acc.at[dstv.at[k]], ssem.at[b],
                             add=True)

            @pl.when(g < _NCHUNK - _NBUF)
            def _prefetch():
                pltpu.make_async_copy(rows[b], acc.at[dstv.at[k]],
                                      ssem.at[b]).wait()
                pltpu.async_copy(haug_h.at[srcv.at[k + _NBUF]], rows[b],
                                 gsem.at[b])
                pltpu.make_async_copy(denr[b], dacc.at[dstv.at[k]],
                                      dsem.at[b]).wait()

    for b in range(_NBUF):
        k = _NCHUNK - _NBUF + b
        pltpu.make_async_copy(rows[b], acc.at[dstv.at[k]], ssem.at[b]).wait()
        pltpu.make_async_copy(denr[b], dacc.at[dstv.at[k]], dsem.at[b]).wait()

    plsc.subcore_barrier()
    drains = []
    for z in range(_RPT // _ZR):
        r0 = s * _RPT + z * _ZR
        drains.append((acc.at[pl.ds(r0, _ZR)], out_h.at[c, pl.ds(r0, _ZR)]))
        drains.append((dacc.at[pl.ds(r0, _ZR)], den_h.at[c, pl.ds(r0, _ZR)]))
    for src, dst in drains:
        pltpu.async_copy(src, dst, gsem.at[0])
    for src, dst in drains:
        pltpu.make_async_copy(src, dst, gsem.at[0]).wait()


_sc_edge = functools.partial(
    pl.kernel,
    out_type=(jax.ShapeDtypeStruct((_NC, _NP, _AW), jnp.float32),
              jax.ShapeDtypeStruct((_NC, _NP, _DW), jnp.float32)),
    mesh=plsc.VectorSubcoreMesh(core_axis_name="c", subcore_axis_name="s"),
    compiler_params=pltpu.CompilerParams(needs_layout_passes=False,
                                         use_tc_tiling_on_sc=False),
    scratch_types=[
        pltpu.VMEM((_N,), jnp.float32),
        pltpu.VMEM((_N,), jnp.float32),
        pltpu.VMEM((2, 16), jnp.float32),
        pltpu.VMEM((_NCHUNK, _CB), jnp.int32),
        pltpu.VMEM((_NCHUNK, _CB), jnp.int32),
        [pltpu.VMEM((_CB, _DW), jnp.float32)] * _NBUF,
        [pltpu.VMEM((_CB, _AW), jnp.float32)] * _NBUF,
        pltpu.VMEM_SHARED((_NP, _AW), jnp.float32),
        pltpu.VMEM_SHARED((_NP, _DW), jnp.float32),
        pltpu.SemaphoreType.DMA((_NBUF,)),
        pltpu.SemaphoreType.DMA((_NBUF,)),
        pltpu.SemaphoreType.DMA((_NBUF,)),
    ],
)(_sc_edge_body)


def kernel(x, edge_index, lin_in_W, lin_in_b, h_lin_W, h_lin_b, lin_W, lin_b,
           gat_W, att_src, att_dst, ln_w, ln_b, betas, pred_W, pred_b):
    src3 = edge_index[0].reshape(_NT, _NCHUNK, _CB)
    dst3 = edge_index[1].reshape(_NT, _NCHUNK, _CB)
    f32 = jnp.float32

    bs_nd = pl.BlockSpec((_NB, _D), lambda i: (i, 0))
    bs_aa = pl.BlockSpec((1, 1, 2 * _NB), lambda i: (i, 0, 0))
    bs_p = pl.BlockSpec((_NC, _NB, _AW), lambda i: (0, i, 0))
    bs_dp = pl.BlockSpec((_NC, _NB, _DW), lambda i: (0, i, 0))
    bs_w = pl.BlockSpec((_D, _D), lambda i: (0, 0))
    bs_v = pl.BlockSpec((_D,), lambda i: (0,))
    bs_m = pl.BlockSpec((2, 16), lambda i: (0, 0))

    node_shapes = [
        jax.ShapeDtypeStruct((_N, _D), f32),
        jax.ShapeDtypeStruct((_NG, 1, 2 * _NB), f32),
        jax.ShapeDtypeStruct((_N, _D), f32),
        jax.ShapeDtypeStruct((_N, _D), f32),
        jax.ShapeDtypeStruct((2, 16), f32),
    ]
    node_specs = [bs_nd, bs_aa, bs_nd, bs_nd, bs_m]
    pro_w_specs = [bs_w, bs_v, bs_v, bs_w, bs_v, bs_w, bs_v]

    haug, aa, hpre, lin, m, _ = pl.pallas_call(
        _first_body,
        grid=(_NG,),
        in_specs=[pl.BlockSpec((_NB, _IN), lambda i: (i, 0)),
                  pl.BlockSpec((_IN, _D), lambda i: (0, 0)), bs_v]
                 + pro_w_specs,
        out_specs=node_specs + [bs_nd],
        out_shape=node_shapes + [jax.ShapeDtypeStruct((_N, _D), f32)],
        compiler_params=_TC_PARAMS,
    )(x, lin_in_W, lin_in_b, gat_W[0], att_src[0].reshape(-1),
      att_dst[0].reshape(-1), h_lin_W[0], h_lin_b[0], lin_W[0], lin_b[0])

    xl = hpre
    for i in range(_L - 1):
        p, dp = _sc_edge(aa, m, src3, dst3, haug)
        haug, aa, hpre, lin, m, xl = pl.pallas_call(
            _mid_bodies[0 if i == 0 else 1],
            grid=(_NG,),
            in_specs=[bs_p, bs_dp, bs_nd, bs_nd, bs_v, bs_v, bs_v, bs_nd]
                     + pro_w_specs,
            out_specs=node_specs + [bs_nd],
            out_shape=node_shapes + [jax.ShapeDtypeStruct((_N, _D), f32)],
            compiler_params=_TC_PARAMS,
        )(p, dp, hpre, lin, ln_w[i], ln_b[i], betas[i], xl,
          gat_W[i + 1], att_src[i + 1].reshape(-1), att_dst[i + 1].reshape(-1),
          h_lin_W[i + 1], h_lin_b[i + 1], lin_W[i + 1], lin_b[i + 1])

    p, dp = _sc_edge(aa, m, src3, dst3, haug)
    out = pl.pallas_call(
        _last_body,
        grid=(_NG,),
        in_specs=[bs_p, bs_dp, bs_nd, bs_nd, bs_v, bs_v, bs_v, bs_nd,
                  pl.BlockSpec((_D, _OUT), lambda i: (0, 0)),
                  pl.BlockSpec((_OUT,), lambda i: (0,))],
        out_specs=pl.BlockSpec((_NB, _OUT), lambda i: (i, 0)),
        out_shape=jax.ShapeDtypeStruct((_N, _OUT), f32),
        compiler_params=_TC_PARAMS,
    )(p, dp, hpre, lin, ln_w[_L - 1], ln_b[_L - 1], betas[_L - 1], xl,
      pred_W, pred_b)
    return out
